# Initial kernel scaffold; baseline (speedup 1.0000x reference)
import jax
import jax.numpy as jnp
from jax.experimental import pallas as pl


def kernel(x, edge_index, edge_attr, batch, fp, fp_length, ws0, wn0, we0, b0, g0, bt0, ws1, wn1, we1, b1, g1, bt1, ws2, wn2, we2, b2, g2, bt2, lw0, lb0, lw1, lb1, lw2, lb2, ow, ob):
    raise NotImplementedError("write your pallas kernel here")


# speedup vs baseline: 3.1943x; 3.1943x over previous
import functools

import jax
import jax.numpy as jnp
from jax import lax
from jax.experimental import pallas as pl
from jax.experimental.pallas import tpu as pltpu
from jax.experimental.pallas import tpu_sc as plsc

N = 10000
E = 320000
ED = 16
G = 256

NC = 2
NS = 16
NW = NC * NS
EPT = E // NW
CH = 80
NCHUNK = EPT // CH
NP = 10240
RPT = NP // NS
BLK = 1024
NBLK = NP // BLK
F = 128

_MESH = plsc.VectorSubcoreMesh(core_axis_name="c", subcore_axis_name="s")


def _edge_pass_body(gather, src_hbm, dst_hbm, dat_hbm, z_hbm, agg_out,
                    srcv, dstv, rows_v, agg_s, sem):
    c = lax.axis_index("c")
    s = lax.axis_index("s")
    wid = c * NS + s
    rbase = pl.multiple_of(s * RPT, 8)

    pltpu.sync_copy(z_hbm, rows_v)
    for j in range(RPT // CH):
        o = pl.multiple_of(rbase + j * CH, 8)
        pltpu.sync_copy(rows_v, agg_s.at[pl.ds(o, CH)])
    plsc.subcore_barrier()

    def chunk(k, _):
        e0 = pl.multiple_of(wid * EPT + k * CH, 8)
        pltpu.sync_copy(dst_hbm.at[pl.ds(e0, CH)], dstv)
        if gather:
            pltpu.sync_copy(src_hbm.at[pl.ds(e0, CH)], srcv)
            pltpu.async_copy(dat_hbm.at[srcv], rows_v, sem).wait()
        else:
            pltpu.sync_copy(dat_hbm.at[pl.ds(e0, CH)], rows_v)
        pltpu.sync_copy(rows_v, agg_s.at[dstv], add=True)
        return 0

    lax.fori_loop(0, NCHUNK, chunk, 0)
    plsc.subcore_barrier()

    obase = pl.multiple_of(c * NP + s * RPT, 8)
    for j in range(RPT // CH):
        oi = pl.multiple_of(rbase + j * CH, 8)
        oo = pl.multiple_of(obase + j * CH, 8)
        pltpu.sync_copy(agg_s.at[pl.ds(oi, CH)], rows_v)
        pltpu.sync_copy(rows_v, agg_out.at[pl.ds(oo, CH)])


def _make_edge_pass(gather):
    def body(src, dst, dat, z, agg_out, *sc):
        _edge_pass_body(gather, src, dst, dat, z, agg_out, *sc)

    return pl.kernel(
        body,
        out_type=[jax.ShapeDtypeStruct((NC * NP, F), jnp.float32)],
        mesh=_MESH,
        scratch_types=[
            pltpu.VMEM((CH,), jnp.int32),
            pltpu.VMEM((CH,), jnp.int32),
            pltpu.VMEM((CH, F), jnp.float32),
            pltpu.VMEM_SHARED((NP, F), jnp.float32),
            pltpu.SemaphoreType.DMA,
        ])


_edge_gather_pass = _make_edge_pass(True)
_edge_linear_pass = _make_edge_pass(False)

_DOT = functools.partial(jnp.dot, preferred_element_type=jnp.float32,
                         precision=jax.lax.Precision.HIGHEST)


def _row_mask(i):
    rid = i * BLK + jax.lax.broadcasted_iota(jnp.int32, (BLK, 1), 0)
    return rid < N


def _tc_a(x_ref, wn_ref, ws_ref, hn_ref, hs_ref):
    x = x_ref[...]
    hn_ref[...] = _DOT(x, wn_ref[...])
    hs_ref[...] = _DOT(x, ws_ref[...])


def _tc_stats(fin):
    def body(hs_ref, agga_ref, aggb_ref, eaa_ref, eab_ref, we_ref, b_ref,
             hraw_ref, ea_ref, sums_ref, sumsq_ref, acc1, acc2):
        i = pl.program_id(0)
        ea = eaa_ref[:, 0:ED] + eab_ref[:, 0:ED]
        t = (hs_ref[...] + agga_ref[:, 0:fin] + aggb_ref[:, 0:fin]
             + _DOT(ea, we_ref[...]) + b_ref[...])
        h = jnp.where(_row_mask(i), jnp.maximum(t, 0.0), 0.0)
        hraw_ref[...] = h
        ea_ref[...] = ea

        @pl.when(i == 0)
        def _():
            acc1[...] = jnp.zeros_like(acc1)
            acc2[...] = jnp.zeros_like(acc2)

        acc1[...] += jnp.sum(h, axis=0, keepdims=True)
        acc2[...] += jnp.sum(h * h, axis=0, keepdims=True)

        @pl.when(i == NBLK - 1)
        def _():
            sums_ref[...] = acc1[...]
            sumsq_ref[...] = acc2[...]

    return body


def _tc_norm(hraw_ref, sums_ref, sumsq_ref, g_ref, bt_ref, wn_ref, ws_ref,
             hn_ref, hs_ref):
    i = pl.program_id(0)
    mean = sums_ref[...] / N
    var = sumsq_ref[...] / N - mean * mean
    scale = jax.lax.rsqrt(var + 1e-5) * g_ref[...]
    h = jnp.where(_row_mask(i),
                  (hraw_ref[...] - mean) * scale + bt_ref[...], 0.0)
    hn_ref[...] = _DOT(h, wn_ref[...])
    hs_ref[...] = _DOT(h, ws_ref[...])


def _tc_d(hs_ref, agg_ref, ea_ref, we_ref, b_ref, g_ref, bt_ref, batch_ref,
          lw0_ref, lb0_ref, lw1_ref, lb1_ref, lw2_ref, lb2_ref, ow_ref, ob_ref,
          out_ref):
    t = (hs_ref[...] + agg_ref[0:NP, 0:16] + agg_ref[NP:2 * NP, 0:16]
         + _DOT(ea_ref[...], we_ref[...]) + b_ref[...])[0:N, :]
    h = jnp.maximum(t, 0.0)
    total = jnp.sum(h)
    mean = total / N
    var = jnp.sum(h * h) / N - mean * mean
    scale = g_ref[0, 0] * jax.lax.rsqrt(var + 1e-5)
    shift = bt_ref[0, 0] - mean * scale
    col = jax.lax.broadcasted_iota(jnp.int32, (N, 16), 1)
    haug = h + jnp.where(col == 1, 1.0, 0.0)
    gids = jax.lax.broadcasted_iota(jnp.int32, (G, 1), 0)
    mask = (batch_ref[...] == gids).astype(jnp.float32)
    z = _DOT(mask, haug)
    y = z[:, 0:1] * scale + z[:, 1:2] * shift
    y = jnp.maximum(_DOT(y, lw0_ref[...]) + lb0_ref[...], 0.0)
    y = jnp.maximum(_DOT(y, lw1_ref[...]) + lb1_ref[...], 0.0)
    y = jnp.maximum(_DOT(y, lw2_ref[...]) + lb2_ref[...], 0.0)
    out_ref[...] = _DOT(y, ow_ref[...]) + ob_ref[...]


def _rows(i):
    return (i, 0)


def _rows_hi(i):
    return (i + NBLK, 0)


def _rep(i):
    return (0, 0)


def _blocked_layer(fin, fout, hs, agg, ea128, ea, we, b, g, bt, wn, ws,
                   first_layer):
    f32 = jnp.float32
    sds = jax.ShapeDtypeStruct
    bspec = functools.partial(pl.BlockSpec, index_map=_rows)
    rep = functools.partial(pl.BlockSpec, index_map=_rep)

    eaw = F if first_layer else ED
    in_specs = [
        bspec((BLK, fin)),
        bspec((BLK, F)),
        pl.BlockSpec((BLK, F), index_map=_rows_hi),
        bspec((BLK, eaw)),
        (pl.BlockSpec((BLK, eaw), index_map=_rows_hi) if first_layer
         else bspec((BLK, eaw))),
        rep((ED, fin)),
        rep((1, fin)),
    ]
    outs = [sds((NP, fin), f32), sds((NP, ED), f32),
            sds((1, fin), f32), sds((1, fin), f32)]
    out_specs = [bspec((BLK, fin)), bspec((BLK, ED)),
                 rep((1, fin)), rep((1, fin))]
    ea_a = ea128 if first_layer else ea
    ea_b = ea128 if first_layer else jnp.zeros_like(ea)
    hraw, ea_out, sums, sumsq = pl.pallas_call(
        _tc_stats(fin), grid=(NBLK,),
        in_specs=in_specs, out_specs=out_specs, out_shape=outs,
        scratch_shapes=[pltpu.VMEM((1, fin), f32), pltpu.VMEM((1, fin), f32)],
    )(hs, agg, agg, ea_a, ea_b, we, b)

    hn, hs_next = pl.pallas_call(
        _tc_norm, grid=(NBLK,),
        in_specs=[bspec((BLK, fin)), rep((1, fin)), rep((1, fin)),
                  rep((1, fin)), rep((1, fin)),
                  rep((fin, F)), rep((fin, fout))],
        out_specs=[bspec((BLK, F)), bspec((BLK, fout))],
        out_shape=[sds((NP, F), f32), sds((NP, fout), f32)],
    )(hraw, sums, sumsq, g, bt, wn, ws)
    return hn, hs_next, ea_out


def _padw(w, cols):
    return jnp.concatenate(
        [w, jnp.zeros((w.shape[0], cols - w.shape[1]), jnp.float32)], axis=1)


def kernel(x, edge_index, edge_attr, batch, fp, fp_length,
           ws0, wn0, we0, b0, g0, bt0,
           ws1, wn1, we1, b1, g1, bt1,
           ws2, wn2, we2, b2, g2, bt2,
           lw0, lb0, lw1, lb1, lw2, lb2, ow, ob):
    f32 = jnp.float32
    src = edge_index[0]
    dst = edge_index[1]
    xp = jnp.pad(x, ((0, NP - N), (0, 0)))
    eap = jnp.pad(edge_attr, ((0, 0), (0, F - ED)))
    wn1p = _padw(wn1, F)
    wn2p = _padw(wn2, F)
    ws2p = _padw(ws2, 16)
    we2p = _padw(we2, 16)
    b2p = _padw(b2.reshape(1, 1), 16)
    z128 = jnp.zeros((CH, F), f32)

    sds = jax.ShapeDtypeStruct
    bspec = functools.partial(pl.BlockSpec, index_map=_rows)
    rep = functools.partial(pl.BlockSpec, index_map=_rep)

    ea128, = _edge_linear_pass(src, dst, eap, z128)

    hn0, hs0 = pl.pallas_call(
        _tc_a, grid=(NBLK,),
        in_specs=[bspec((BLK, 128)), rep((128, 128)), rep((128, 128))],
        out_specs=[bspec((BLK, 128)), bspec((BLK, 128))],
        out_shape=[sds((NP, 128), f32), sds((NP, 128), f32)],
    )(xp, wn0, ws0)

    agg0, = _edge_gather_pass(src, dst, hn0, z128)

    hn1, hs1, ea = _blocked_layer(
        128, 64, hs0, agg0, ea128, None, we0, b0.reshape(1, -1),
        g0.reshape(1, -1), bt0.reshape(1, -1), wn1p, ws1, first_layer=True)

    agg1, = _edge_gather_pass(src, dst, hn1, z128)

    hn2, hs2, _ = _blocked_layer(
        64, 16, hs1, agg1, None, ea, we1, b1.reshape(1, -1),
        g1.reshape(1, -1), bt1.reshape(1, -1), wn2p, ws2p, first_layer=False)

    agg2, = _edge_gather_pass(src, dst, hn2, z128)

    out = pl.pallas_call(
        _tc_d, out_shape=sds((G, 1), f32))(
        hs2, agg2, ea, we2p, b2p, g2.reshape(1, -1), bt2.reshape(1, -1),
        batch.reshape(1, N), lw0, lb0.reshape(1, -1), lw1, lb1.reshape(1, -1),
        lw2, lb2.reshape(1, -1), ow, ob.reshape(1, -1))
    return out
